# baseline (speedup 1.0000x reference)
import functools

import jax
import jax.numpy as jnp
from jax import lax
from jax.experimental import pallas as pl
from jax.experimental.pallas import tpu as pltpu
from jax.experimental.pallas import tpu_sc as plsc

_N = 12288
_D = 192
_DH = _D // 2
_L = 16

_info = plsc.get_sparse_core_info()
_NC = _info.num_cores
_NS = _info.num_subcores
_NW = _NC * _NS
_TOK = _N // _NW
_C = 96
_NCHUNK = _TOK // _C
_DP = 256
_DHP = 128

_MAX_IDX = 1023


def _pe_body(scale_t, orient_t, spatial_t, sf, scales, orients, pos_h, pos_w,
             out,
             sf_v, idx_s, idx_o, idx_h, idx_w,
             s_rows, o_rows, h_rows, w_rows, acc, sem):
    wid = lax.axis_index("s") * _NC + lax.axis_index("c")
    pltpu.sync_copy(sf, sf_v)
    f0 = sf_v[0, :]
    f1 = sf_v[1, :]
    f2 = sf_v[2, :]
    for c in range(_NCHUNK):
        base = wid * _TOK + c * _C
        pltpu.sync_copy(scales.at[pl.ds(base, _C)], idx_s)
        pltpu.sync_copy(orients.at[pl.ds(base, _C)], idx_o)
        pltpu.sync_copy(pos_h.at[pl.ds(base, _C)], idx_h)
        pltpu.sync_copy(pos_w.at[pl.ds(base, _C)], idx_w)
        for i in range(_C // _L):
            sl = pl.ds(i * _L, _L)
            s = idx_s[sl]
            f = jnp.where(s == 0, f0, jnp.where(s == 1, f1, f2))
            h = (idx_h[sl].astype(jnp.float32) * f).astype(jnp.int32)
            idx_h[sl] = jnp.clip(h, 0, _MAX_IDX)
            w = (idx_w[sl].astype(jnp.float32) * f).astype(jnp.int32)
            idx_w[sl] = jnp.clip(w, 0, _MAX_IDX)
        cp_s = pltpu.async_copy(scale_t.at[idx_s], s_rows, sem)
        cp_o = pltpu.async_copy(orient_t.at[idx_o], o_rows, sem)
        cp_h = pltpu.async_copy(spatial_t.at[idx_h], h_rows, sem)
        cp_w = pltpu.async_copy(spatial_t.at[idx_w], w_rows, sem)
        cp_s.wait()
        cp_o.wait()
        cp_h.wait()
        cp_w.wait()

        def _row(r, carry):
            for j in range(_DH // _L):
                slj = pl.ds(j * _L, _L)
                sl2 = pl.ds(_DH + j * _L, _L)
                acc[r, slj] = s_rows[r, slj] + o_rows[r, slj] + h_rows[r, slj]
                acc[r, sl2] = s_rows[r, sl2] + o_rows[r, sl2] + w_rows[r, slj]
            return carry

        lax.fori_loop(0, _C, _row, 0)
        pltpu.sync_copy(acc, out.at[pl.ds(base, _C)])


_pe_kernel = functools.partial(
    pl.kernel,
    mesh=plsc.VectorSubcoreMesh(core_axis_name="c", subcore_axis_name="s"),
    out_type=jax.ShapeDtypeStruct((_N, _D), jnp.float32),
    scratch_types=[
        pltpu.VMEM((3, _L), jnp.float32),
        pltpu.VMEM((_C,), jnp.int32),
        pltpu.VMEM((_C,), jnp.int32),
        pltpu.VMEM((_C,), jnp.int32),
        pltpu.VMEM((_C,), jnp.int32),
        pltpu.VMEM((_C, _DP), jnp.float32),
        pltpu.VMEM((_C, _DP), jnp.float32),
        pltpu.VMEM((_C, _DHP), jnp.float32),
        pltpu.VMEM((_C, _DHP), jnp.float32),
        pltpu.VMEM((_C, _D), jnp.float32),
        pltpu.SemaphoreType.DMA,
    ],
)(_pe_body)


_TN = 1024


def _add_body(tok_ref, pe_ref, out_ref):
    out_ref[...] = tok_ref[...] + pe_ref[...]


def kernel(tokens, scale_table, orient_table, spatial_table, scale_factors,
           scales, orientations, positions):
    B, N, D = tokens.shape
    sf = jnp.broadcast_to(scale_factors[:, None], (3, _L))
    scale_p = jnp.pad(scale_table, ((0, 0), (0, _DP - _D)))
    orient_p = jnp.pad(orient_table, ((0, 0), (0, _DP - _D)))
    spatial_p = jnp.pad(spatial_table, ((0, 0), (0, _DHP - _DH)))
    pos_h = positions[:, 0]
    pos_w = positions[:, 1]
    pe = _pe_kernel(scale_p, orient_p, spatial_p, sf,
                    scales, orientations, pos_h, pos_w)
    out = pl.pallas_call(
        _add_body,
        grid=(N // _TN, B),
        in_specs=[
            pl.BlockSpec((1, _TN, D), lambda n, b: (b, n, 0)),
            pl.BlockSpec((_TN, D), lambda n, b: (n, 0)),
        ],
        out_specs=pl.BlockSpec((1, _TN, D), lambda n, b: (b, n, 0)),
        out_shape=jax.ShapeDtypeStruct((B, N, D), tokens.dtype),
    )(tokens, pe)
    return out


# speedup vs baseline: 1.0033x; 1.0033x over previous
import functools

import jax
import jax.numpy as jnp
from jax import lax
from jax.experimental import pallas as pl
from jax.experimental.pallas import tpu as pltpu
from jax.experimental.pallas import tpu_sc as plsc

_N = 12288
_D = 192
_DH = _D // 2
_L = 16

_info = plsc.get_sparse_core_info()
_NC = _info.num_cores
_NS = _info.num_subcores
_NW = _NC * _NS
_TOK = _N // _NW
_C = 48
_NCHUNK = _TOK // _C
_DP = 256
_DHP = 128

_MAX_IDX = 1023


def _pe_body(scale_t, orient_t, spatial_t, sf, scales, orients, pos_h, pos_w,
             out,
             sf_v, idx_s, idx_o, idx_h, idx_w,
             s_rows, o_rows, h_rows, w_rows, acc, sems):
    wid = lax.axis_index("s") * _NC + lax.axis_index("c")
    base = wid * _TOK
    m0 = pltpu.async_copy(scales.at[pl.ds(base, _TOK)], idx_s, sems.at[0])
    m1 = pltpu.async_copy(orients.at[pl.ds(base, _TOK)], idx_o, sems.at[1])
    m2 = pltpu.async_copy(pos_h.at[pl.ds(base, _TOK)], idx_h, sems.at[0])
    m3 = pltpu.async_copy(pos_w.at[pl.ds(base, _TOK)], idx_w, sems.at[1])
    pltpu.sync_copy(sf, sf_v)
    f0 = sf_v[0, :]
    f1 = sf_v[1, :]
    f2 = sf_v[2, :]
    m0.wait()
    m1.wait()
    m2.wait()
    m3.wait()

    @plsc.parallel_loop(0, _TOK // _L, unroll=2)
    def _idx(i):
        sl = pl.ds(i * _L, _L)
        s = idx_s[sl]
        f = jnp.where(s == 0, f0, jnp.where(s == 1, f1, f2))
        h = (idx_h[sl].astype(jnp.float32) * f).astype(jnp.int32)
        idx_h[sl] = jnp.clip(h, 0, _MAX_IDX)
        w = (idx_w[sl].astype(jnp.float32) * f).astype(jnp.int32)
        idx_w[sl] = jnp.clip(w, 0, _MAX_IDX)

    def _fire(c, buf):
        sl = pl.ds(c * _C, _C)
        sm = sems.at[buf]
        return (pltpu.async_copy(scale_t.at[idx_s.at[sl]], s_rows.at[buf], sm),
                pltpu.async_copy(orient_t.at[idx_o.at[sl]], o_rows.at[buf], sm),
                pltpu.async_copy(spatial_t.at[idx_h.at[sl]], h_rows.at[buf], sm),
                pltpu.async_copy(spatial_t.at[idx_w.at[sl]], w_rows.at[buf], sm))

    cps = _fire(0, 0)
    for c in range(_NCHUNK):
        buf = c % 2
        nxt = (c + 1) % 2
        if c + 1 < _NCHUNK:
            nxt_cps = _fire(c + 1, nxt)
        for cp in cps:
            cp.wait()

        @plsc.parallel_loop(0, _C, unroll=2)
        def _row(r):
            for j in range(_DH // _L):
                slj = pl.ds(j * _L, _L)
                sl2 = pl.ds(_DH + j * _L, _L)
                acc[buf, r, slj] = (s_rows[buf, r, slj] + o_rows[buf, r, slj]
                                    + h_rows[buf, r, slj])
                acc[buf, r, sl2] = (s_rows[buf, r, sl2] + o_rows[buf, r, sl2]
                                    + w_rows[buf, r, slj])

        pltpu.sync_copy(acc.at[buf], out.at[pl.ds(base + c * _C, _C)])
        if c + 1 < _NCHUNK:
            cps = nxt_cps


_pe_kernel = functools.partial(
    pl.kernel,
    mesh=plsc.VectorSubcoreMesh(core_axis_name="c", subcore_axis_name="s"),
    out_type=jax.ShapeDtypeStruct((_N, _D), jnp.float32),
    scratch_types=[
        pltpu.VMEM((3, _L), jnp.float32),
        pltpu.VMEM((_TOK,), jnp.int32),
        pltpu.VMEM((_TOK,), jnp.int32),
        pltpu.VMEM((_TOK,), jnp.int32),
        pltpu.VMEM((_TOK,), jnp.int32),
        pltpu.VMEM((2, _C, _DP), jnp.float32),
        pltpu.VMEM((2, _C, _DP), jnp.float32),
        pltpu.VMEM((2, _C, _DHP), jnp.float32),
        pltpu.VMEM((2, _C, _DHP), jnp.float32),
        pltpu.VMEM((2, _C, _D), jnp.float32),
        pltpu.SemaphoreType.DMA((2,)),
    ],
)(_pe_body)


_TN = 1024


def _add_body(tok_ref, pe_ref, out_ref):
    out_ref[...] = tok_ref[...] + pe_ref[...]


def kernel(tokens, scale_table, orient_table, spatial_table, scale_factors,
           scales, orientations, positions):
    B, N, D = tokens.shape
    sf = jnp.broadcast_to(scale_factors[:, None], (3, _L))
    scale_p = jnp.pad(scale_table, ((0, 0), (0, _DP - _D)))
    orient_p = jnp.pad(orient_table, ((0, 0), (0, _DP - _D)))
    spatial_p = jnp.pad(spatial_table, ((0, 0), (0, _DHP - _DH)))
    pos_h = positions[:, 0]
    pos_w = positions[:, 1]
    pe = _pe_kernel(scale_p, orient_p, spatial_p, sf,
                    scales, orientations, pos_h, pos_w)
    out = pl.pallas_call(
        _add_body,
        grid=(N // _TN, B),
        in_specs=[
            pl.BlockSpec((1, _TN, D), lambda n, b: (b, n, 0)),
            pl.BlockSpec((_TN, D), lambda n, b: (n, 0)),
        ],
        out_specs=pl.BlockSpec((1, _TN, D), lambda n, b: (b, n, 0)),
        out_shape=jax.ShapeDtypeStruct((B, N, D), tokens.dtype),
    )(tokens, pe)
    return out
